# baseline (speedup 1.0000x reference)
import functools

import jax
import jax.numpy as jnp
from jax.experimental import pallas as pl
from jax.experimental.pallas import tpu as pltpu

_N_ACTIONS = 64
_HW = 256
_BIAS_ROW0 = 1280


def _ac_kernel(n_actions, x_ref, w_ref, b_ref, pol_ref, val_ref):
    w1 = w_ref[0:256, :]
    w2a = w_ref[256:512, 0:256]
    w2c = w_ref[512:768, 256:512]
    woa = w_ref[768:1024, 0:128]
    woc = w_ref[1024:1280, 0:128]

    b1 = b_ref[0:1, :]
    b2 = b_ref[8:9, :]
    bo = b_ref[16:17, 0:128]

    x = x_ref[...].astype(jnp.bfloat16)

    h1 = jnp.tanh(jnp.dot(x, w1, preferred_element_type=jnp.float32) + b1)
    h1 = h1.astype(jnp.bfloat16)

    h2a = jnp.tanh(jnp.dot(h1[:, 0:256], w2a,
                           preferred_element_type=jnp.float32) + b2[:, 0:256])
    h2c = jnp.tanh(jnp.dot(h1[:, 256:512], w2c,
                           preferred_element_type=jnp.float32) + b2[:, 256:512])

    o = (jnp.dot(h2a.astype(jnp.bfloat16), woa, preferred_element_type=jnp.float32)
         + jnp.dot(h2c.astype(jnp.bfloat16), woc, preferred_element_type=jnp.float32)
         + bo)

    lane = jax.lax.broadcasted_iota(jnp.int32, o.shape, 1)
    is_logit = lane < n_actions
    masked = jnp.where(is_logit, o, -1e30)
    m = jnp.max(masked, axis=-1, keepdims=True)
    e = jnp.exp(masked - m)
    policy = e / jnp.sum(e, axis=-1, keepdims=True)

    pol_ref[...] = policy[:, :n_actions]
    val_ref[...] = o[:, n_actions:n_actions + 1]


def kernel(states, slab):
    B, in_dim = states.shape
    n_actions = _N_ACTIONS

    w_bf16 = slab[:_BIAS_ROW0].astype(jnp.bfloat16)
    biases = slab[_BIAS_ROW0:]

    block_b = min(B, 2048)
    grid = (pl.cdiv(B, block_b),)

    pol, val = pl.pallas_call(
        functools.partial(_ac_kernel, n_actions),
        out_shape=(
            jax.ShapeDtypeStruct((B, n_actions), jnp.float32),
            jax.ShapeDtypeStruct((B, 1), jnp.float32),
        ),
        grid=grid,
        in_specs=[
            pl.BlockSpec((block_b, in_dim), lambda i: (i, 0)),
            pl.BlockSpec(w_bf16.shape, lambda i: (0, 0)),
            pl.BlockSpec(biases.shape, lambda i: (0, 0)),
        ],
        out_specs=(
            pl.BlockSpec((block_b, n_actions), lambda i: (i, 0)),
            pl.BlockSpec((block_b, 1), lambda i: (i, 0)),
        ),
        compiler_params=pltpu.CompilerParams(
            dimension_semantics=("parallel",)),
    )(states, w_bf16, biases)

    return pol, val


# speedup vs baseline: 1.5868x; 1.5868x over previous
import functools

import jax
import jax.numpy as jnp
from jax.experimental import pallas as pl
from jax.experimental.pallas import tpu as pltpu

_N_ACTIONS = 64
_HW = 256
_BIAS_ROW0 = 1280


def _ac_kernel(n_actions, x_ref, w_ref, b_ref, out_ref):
    w1 = w_ref[0:256, :]
    w2a = w_ref[256:512, 0:256]
    w2c = w_ref[512:768, 256:512]
    woa = w_ref[768:1024, 0:128]
    woc = w_ref[1024:1280, 0:128]

    b1 = b_ref[0:1, :]
    b2 = b_ref[8:9, :]
    bo = b_ref[16:17, 0:128]

    x = x_ref[...].astype(jnp.bfloat16)

    h1 = jnp.tanh(jnp.dot(x, w1, preferred_element_type=jnp.float32) + b1)
    h1 = h1.astype(jnp.bfloat16)

    h2a = jnp.tanh(jnp.dot(h1[:, 0:256], w2a,
                           preferred_element_type=jnp.float32) + b2[:, 0:256])
    h2c = jnp.tanh(jnp.dot(h1[:, 256:512], w2c,
                           preferred_element_type=jnp.float32) + b2[:, 256:512])

    o = (jnp.dot(h2a.astype(jnp.bfloat16), woa, preferred_element_type=jnp.float32)
         + jnp.dot(h2c.astype(jnp.bfloat16), woc, preferred_element_type=jnp.float32)
         + bo)

    out_w = out_ref.shape[-1]
    o = o[:, :out_w]
    lane = jax.lax.broadcasted_iota(jnp.int32, o.shape, 1)
    is_logit = lane < n_actions
    masked = jnp.where(is_logit, o, -1e30)
    m = jnp.max(masked, axis=-1, keepdims=True)
    e = jnp.exp(masked - m)
    policy = e / jnp.sum(e, axis=-1, keepdims=True)

    merged = jnp.where(is_logit, policy,
                       jnp.where(lane == n_actions, o, 0.0))
    out_ref[...] = merged


def kernel(states, slab):
    B, in_dim = states.shape
    n_actions = _N_ACTIONS

    w_bf16 = slab[:_BIAS_ROW0].astype(jnp.bfloat16)
    biases = slab[_BIAS_ROW0:]

    block_b = min(B, 4096)
    grid = (pl.cdiv(B, block_b),)
    out_w = 72

    out = pl.pallas_call(
        functools.partial(_ac_kernel, n_actions),
        out_shape=jax.ShapeDtypeStruct((B, out_w), jnp.float32),
        grid=grid,
        in_specs=[
            pl.BlockSpec((block_b, in_dim), lambda i: (i, 0)),
            pl.BlockSpec(w_bf16.shape, lambda i: (0, 0)),
            pl.BlockSpec(biases.shape, lambda i: (0, 0)),
        ],
        out_specs=pl.BlockSpec((block_b, out_w), lambda i: (i, 0)),
        compiler_params=pltpu.CompilerParams(
            dimension_semantics=("parallel",)),
    )(states, w_bf16, biases)

    return out[:, :n_actions], out[:, n_actions:n_actions + 1]


# speedup vs baseline: 1.8908x; 1.1916x over previous
import functools

import jax
import jax.numpy as jnp
from jax.experimental import pallas as pl
from jax.experimental.pallas import tpu as pltpu

_N_ACTIONS = 64
_HW = 256
_BIAS_ROW0 = 1280


def _ac_kernel(n_actions, x_ref, w_ref, wv_ref, b_ref, pol_ref, val_ref):
    w1 = w_ref[0:256, :]
    w2a = w_ref[256:512, 0:256]
    w2c = w_ref[512:768, 256:512]
    woa = w_ref[768:1024, 0:128]

    b1 = b_ref[0:1, :]
    b2 = b_ref[8:9, :]
    bo = b_ref[16:17, 0:128]

    x = x_ref[...].astype(jnp.bfloat16)

    h1 = jnp.tanh(jnp.dot(x, w1, preferred_element_type=jnp.float32) + b1)
    h1 = h1.astype(jnp.bfloat16)

    h2a = jnp.tanh(jnp.dot(h1[:, 0:256], w2a,
                           preferred_element_type=jnp.float32) + b2[:, 0:256])
    h2c = jnp.tanh(jnp.dot(h1[:, 256:512], w2c,
                           preferred_element_type=jnp.float32) + b2[:, 256:512])

    o = jnp.dot(h2a.astype(jnp.bfloat16), woa,
                preferred_element_type=jnp.float32) + bo

    lane = jax.lax.broadcasted_iota(jnp.int32, o.shape, 1)
    is_logit = lane < n_actions
    masked = jnp.where(is_logit, o, -1e30)
    m = jnp.max(masked, axis=-1, keepdims=True)
    e = jnp.exp(masked - m)
    policy = e / jnp.sum(e, axis=-1, keepdims=True)
    pol_ref[...] = policy[:, :n_actions]

    vo = jnp.dot(h2c.astype(jnp.bfloat16), wv_ref[...],
                 preferred_element_type=jnp.float32)
    val_ref[...] = vo[:, 0:1] + b_ref[16:17, 64:65]


def kernel(states, slab):
    B, in_dim = states.shape
    n_actions = _N_ACTIONS

    w_bf16 = slab[:1024].astype(jnp.bfloat16)
    wv = jnp.pad(slab[1024:_BIAS_ROW0, 64:65],
                 ((0, 0), (0, 127))).astype(jnp.bfloat16)
    biases = slab[_BIAS_ROW0:]

    block_b = min(B, 4096)
    grid = (pl.cdiv(B, block_b),)

    pol, val = pl.pallas_call(
        functools.partial(_ac_kernel, n_actions),
        out_shape=(
            jax.ShapeDtypeStruct((B, n_actions), jnp.float32),
            jax.ShapeDtypeStruct((B, 1), jnp.float32),
        ),
        grid=grid,
        in_specs=[
            pl.BlockSpec((block_b, in_dim), lambda i: (i, 0)),
            pl.BlockSpec(w_bf16.shape, lambda i: (0, 0)),
            pl.BlockSpec(wv.shape, lambda i: (0, 0)),
            pl.BlockSpec(biases.shape, lambda i: (0, 0)),
        ],
        out_specs=(
            pl.BlockSpec((block_b, n_actions), lambda i: (i, 0)),
            pl.BlockSpec((block_b, 1), lambda i: (i, 0)),
        ),
        compiler_params=pltpu.CompilerParams(
            dimension_semantics=("parallel",)),
    )(states, w_bf16, wv, biases)

    return pol, val
